# Initial kernel scaffold; baseline (speedup 1.0000x reference)
import jax
import jax.numpy as jnp
from jax.experimental import pallas as pl


def kernel(x, W_enc, b_enc, W_dec, b_dec):
    raise NotImplementedError("write your pallas kernel here")


# speedup vs baseline: 8.1547x; 8.1547x over previous
import jax
import jax.numpy as jnp
from jax import lax
from jax.experimental import pallas as pl

ACT = 2048
DICT = 16384
K = 64
BATCH = 4096

BT_ENC = 256
FT = 2048
BT_TH = 128
BT_DEC = 256


def _encode_kernel(x_ref, w_ref, be_ref, bd_ref, out_ref):
    xt = x_ref[...] - bd_ref[0]
    acc = lax.dot_general(xt, w_ref[...], (((1,), (1,)), ((), ())),
                          preferred_element_type=jnp.float32)
    out_ref[...] = jnp.maximum(acc + be_ref[0, 0], 0.0)


def _threshold_kernel(v_ref, out_ref):
    v = v_ref[...]
    bits = lax.bitcast_convert_type(v, jnp.int32)

    def body(b, T):
        cand = T | (1 << (30 - b))
        cnt = jnp.sum((bits >= cand).astype(jnp.int32), axis=1, keepdims=True)
        return jnp.where(cnt >= K, cand, T)

    T = lax.fori_loop(0, 31, body, jnp.zeros((v.shape[0], 1), jnp.int32))
    out_ref[...] = jnp.where(bits >= T, v, 0.0)


def _decode_kernel(enc_ref, w_ref, bd_ref, out_ref):
    j = pl.program_id(1)
    acc = lax.dot_general(enc_ref[...], w_ref[...], (((1,), (1,)), ((), ())),
                          preferred_element_type=jnp.float32)

    @pl.when(j == 0)
    def _():
        out_ref[...] = acc + bd_ref[0]

    @pl.when(j != 0)
    def _():
        out_ref[...] = out_ref[...] + acc


def kernel(x, W_enc, b_enc, W_dec, b_dec):
    be2 = b_enc.reshape(DICT // FT, 1, FT)
    bd2 = b_dec.reshape(1, ACT)

    post_relu = pl.pallas_call(
        _encode_kernel,
        grid=(BATCH // BT_ENC, DICT // FT),
        in_specs=[
            pl.BlockSpec((BT_ENC, ACT), lambda i, j: (i, 0)),
            pl.BlockSpec((FT, ACT), lambda i, j: (j, 0)),
            pl.BlockSpec((1, 1, FT), lambda i, j: (j, 0, 0)),
            pl.BlockSpec((1, ACT), lambda i, j: (0, 0)),
        ],
        out_specs=pl.BlockSpec((BT_ENC, FT), lambda i, j: (i, j)),
        out_shape=jax.ShapeDtypeStruct((BATCH, DICT), jnp.float32),
    )(x, W_enc, be2, bd2)

    encoded = pl.pallas_call(
        _threshold_kernel,
        grid=(BATCH // BT_TH,),
        in_specs=[pl.BlockSpec((BT_TH, DICT), lambda i: (i, 0))],
        out_specs=pl.BlockSpec((BT_TH, DICT), lambda i: (i, 0)),
        out_shape=jax.ShapeDtypeStruct((BATCH, DICT), jnp.float32),
    )(post_relu)

    x_hat = pl.pallas_call(
        _decode_kernel,
        grid=(BATCH // BT_DEC, DICT // FT),
        in_specs=[
            pl.BlockSpec((BT_DEC, FT), lambda i, j: (i, j)),
            pl.BlockSpec((ACT, FT), lambda i, j: (0, j)),
            pl.BlockSpec((1, ACT), lambda i, j: (0, 0)),
        ],
        out_specs=pl.BlockSpec((BT_DEC, ACT), lambda i, j: (i, 0)),
        out_shape=jax.ShapeDtypeStruct((BATCH, ACT), jnp.float32),
    )(encoded, W_dec, bd2)

    return x_hat


# speedup vs baseline: 11.4600x; 1.4053x over previous
import jax
import jax.numpy as jnp
from jax import lax
from jax.experimental import pallas as pl

ACT = 2048
DICT = 16384
K = 64
BATCH = 4096

BT_ENC = 256
FT = 2048
BT_TH = 128
BT_DEC = 1024
KT_DEC = 1024


def _encode_kernel(x_ref, w_ref, be_ref, bd_ref, out_ref):
    xt = x_ref[...] - bd_ref[0]
    acc = lax.dot_general(xt, w_ref[...], (((1,), (1,)), ((), ())),
                          preferred_element_type=jnp.float32)
    out_ref[...] = jnp.maximum(acc + be_ref[0, 0], 0.0)


def _threshold_kernel(v_ref, out_ref):
    v = v_ref[...]
    bits = lax.bitcast_convert_type(v, jnp.int32)

    def body(b, T):
        cand = T | (1 << (30 - b))
        cnt = jnp.sum((bits >= cand).astype(jnp.int32), axis=1, keepdims=True)
        return jnp.where(cnt >= K, cand, T)

    T = lax.fori_loop(0, 31, body, jnp.zeros((v.shape[0], 1), jnp.int32))
    out_ref[...] = jnp.where(bits >= T, v, 0.0)


def _decode_kernel(enc_ref, w_ref, bd_ref, out_ref):
    j = pl.program_id(1)
    acc = lax.dot_general(enc_ref[...], w_ref[...], (((1,), (1,)), ((), ())),
                          preferred_element_type=jnp.float32)

    @pl.when(j == 0)
    def _():
        out_ref[...] = acc + bd_ref[0]

    @pl.when(j != 0)
    def _():
        out_ref[...] = out_ref[...] + acc


def kernel(x, W_enc, b_enc, W_dec, b_dec):
    be2 = b_enc.reshape(DICT // FT, 1, FT)
    bd2 = b_dec.reshape(1, ACT)

    post_relu = pl.pallas_call(
        _encode_kernel,
        grid=(DICT // FT, BATCH // BT_ENC),
        in_specs=[
            pl.BlockSpec((BT_ENC, ACT), lambda j, i: (i, 0)),
            pl.BlockSpec((FT, ACT), lambda j, i: (j, 0)),
            pl.BlockSpec((1, 1, FT), lambda j, i: (j, 0, 0)),
            pl.BlockSpec((1, ACT), lambda j, i: (0, 0)),
        ],
        out_specs=pl.BlockSpec((BT_ENC, FT), lambda j, i: (i, j)),
        out_shape=jax.ShapeDtypeStruct((BATCH, DICT), jnp.float32),
    )(x, W_enc, be2, bd2)

    encoded = pl.pallas_call(
        _threshold_kernel,
        grid=(BATCH // BT_TH,),
        in_specs=[pl.BlockSpec((BT_TH, DICT), lambda i: (i, 0))],
        out_specs=pl.BlockSpec((BT_TH, DICT), lambda i: (i, 0)),
        out_shape=jax.ShapeDtypeStruct((BATCH, DICT), jnp.float32),
    )(post_relu)

    x_hat = pl.pallas_call(
        _decode_kernel,
        grid=(BATCH // BT_DEC, DICT // KT_DEC),
        in_specs=[
            pl.BlockSpec((BT_DEC, KT_DEC), lambda i, j: (i, j)),
            pl.BlockSpec((ACT, KT_DEC), lambda i, j: (0, j)),
            pl.BlockSpec((1, ACT), lambda i, j: (0, 0)),
        ],
        out_specs=pl.BlockSpec((BT_DEC, ACT), lambda i, j: (i, 0)),
        out_shape=jax.ShapeDtypeStruct((BATCH, ACT), jnp.float32),
    )(encoded, W_dec, bd2)

    return x_hat
